# Initial kernel scaffold; baseline (speedup 1.0000x reference)
import jax
import jax.numpy as jnp
from jax.experimental import pallas as pl


def kernel(points, features, W_geom, g_geom, b_geom, W_sem, g_sem, b_sem, W_fuse, g_fuse, b_fuse):
    raise NotImplementedError("write your pallas kernel here")


# speedup vs baseline: 15.6156x; 15.6156x over previous
import functools

import jax
import jax.numpy as jnp
from jax import lax
from jax.experimental import pallas as pl
from jax.experimental.pallas import tpu as pltpu
from jax.experimental.pallas import tpu_sc as plsc

_EPS = 1e-5
_K = 16
_ROW_TILE = 256
_PTILE = 2048


def _knn_body(prow_ref, pcol_ref, idx_ref):
    b = pl.program_id(0)
    n = pcol_ref.shape[2]
    r = prow_ref.shape[1]
    pr = prow_ref[0]
    pc = pcol_ref[0]
    xr, yr, zr = pr[:, 0:1], pr[:, 1:2], pr[:, 2:3]
    xc, yc, zc = pc[0:1, :], pc[1:2, :], pc[2:3, :]
    sq_r = xr * xr + yr * yr + zr * zr
    sq_c = xc * xc + yc * yc + zc * zc
    rnd = lambda v: v.astype(jnp.bfloat16).astype(jnp.float32)
    cross = (rnd(xr) * rnd(xc) + rnd(yr) * rnd(yc)
             + rnd(zr) * rnd(zc))
    d = (sq_r + sq_c) - 2.0 * cross
    iota = lax.broadcasted_iota(jnp.int32, (r, n), 1)
    iota16 = lax.broadcasted_iota(jnp.int32, (r, _K), 1)
    acc = jnp.zeros((r, _K), jnp.int32)
    big = jnp.float32(jnp.inf)
    for k in range(_K):
        m = jnp.min(d, axis=1, keepdims=True)
        am = jnp.min(jnp.where(d <= m, iota, n), axis=1, keepdims=True)
        acc = jnp.where(iota16 == k, am, acc)
        d = jnp.where(iota == am, big, d)
    idx_ref[0] = acc + b * n


def _knn(points, points_t):
    b, n, _ = points.shape
    grid = (b, n // _ROW_TILE)
    return pl.pallas_call(
        _knn_body,
        grid=grid,
        in_specs=[
            pl.BlockSpec((1, _ROW_TILE, 3), lambda bi, i: (bi, i, 0)),
            pl.BlockSpec((1, 3, n), lambda bi, i: (bi, 0, 0)),
        ],
        out_specs=pl.BlockSpec((1, _ROW_TILE, _K), lambda bi, i: (bi, i, 0)),
        out_shape=jax.ShapeDtypeStruct((b, n, _K), jnp.int32),
    )(points, points_t)


def _tables_body(pf_ref, wt_ref, wz_ref, t_ref, zc_ref):
    pf = pf_ref[0]
    dn = (((1,), (0,)), ((), ()))
    t_ref[0] = lax.dot_general(pf, wt_ref[...], dn,
                               preferred_element_type=jnp.float32)
    zc_ref[0] = lax.dot_general(pf, wz_ref[...], dn,
                                preferred_element_type=jnp.float32)


def _tables(pf, wt, wz):
    b, n, cin = pf.shape
    w = wt.shape[1]
    t, zc = pl.pallas_call(
        _tables_body,
        grid=(b,),
        in_specs=[
            pl.BlockSpec((1, n, cin), lambda bi: (bi, 0, 0)),
            pl.BlockSpec((cin, w), lambda bi: (0, 0)),
            pl.BlockSpec((cin, w), lambda bi: (0, 0)),
        ],
        out_specs=[
            pl.BlockSpec((1, n, w), lambda bi: (bi, 0, 0)),
            pl.BlockSpec((1, n, w), lambda bi: (bi, 0, 0)),
        ],
        out_shape=[
            jax.ShapeDtypeStruct((b, n, w), jnp.float32),
            jax.ShapeDtypeStruct((b, n, w), jnp.float32),
        ],
    )(pf, wt, wz)
    return t.reshape(b * n, w), zc.reshape(b * n, w)


_GCH = 128


def _sc_gather(table, idx_flat):
    rows, width = idx_flat.shape[0], table.shape[1]
    info = plsc.get_sparse_core_info()
    nw = info.num_cores * info.num_subcores
    per_w = rows // nw
    n_ch = per_w // _GCH
    idx2d = idx_flat.reshape(nw * n_ch, _GCH)
    mesh = plsc.VectorSubcoreMesh(core_axis_name="c", subcore_axis_name="s")

    @functools.partial(
        pl.kernel,
        out_type=jax.ShapeDtypeStruct((rows, width), jnp.float32),
        mesh=mesh,
        scratch_types=[
            pltpu.VMEM((n_ch, _GCH), jnp.int32),
            pltpu.VMEM((_GCH, width), jnp.float32),
            pltpu.VMEM((_GCH, width), jnp.float32),
            pltpu.SemaphoreType.DMA,
            pltpu.SemaphoreType.DMA,
        ],
    )
    def gather_kernel(tbl_hbm, idx_hbm, out_hbm, idx_v, buf0, buf1, sem0, sem1):
        wid = lax.axis_index("s") * info.num_cores + lax.axis_index("c")
        pltpu.sync_copy(idx_hbm.at[pl.ds(wid * n_ch, n_ch)], idx_v)
        base = wid * per_w
        pltpu.async_copy(tbl_hbm.at[idx_v.at[0]], buf0, sem0)

        def body(j2, _):
            j = 2 * j2
            pltpu.async_copy(tbl_hbm.at[idx_v.at[j + 1]], buf1, sem1)
            pltpu.make_async_copy(tbl_hbm.at[idx_v.at[j]], buf0, sem0).wait()
            pltpu.sync_copy(buf0, out_hbm.at[pl.ds(base + j * _GCH, _GCH)])

            @pl.when(j + 2 < n_ch)
            def _():
                pltpu.async_copy(tbl_hbm.at[idx_v.at[j + 2]], buf0, sem0)

            pltpu.make_async_copy(tbl_hbm.at[idx_v.at[j + 1]], buf1,
                                  sem1).wait()
            pltpu.sync_copy(buf1,
                            out_hbm.at[pl.ds(base + (j + 1) * _GCH, _GCH)])
            return 0

        lax.fori_loop(0, n_ch // 2, body, 0)

    return gather_kernel(table, idx2d)


def _ystats_body(g_ref, zc_ref, stats_ref):
    j = pl.program_id(0)
    k = pl.program_id(1)
    y = g_ref[...] + zc_ref[...]
    s = jnp.sum(y, axis=0, keepdims=True)
    q = jnp.sum(y * y, axis=0, keepdims=True)

    @pl.when(jnp.logical_and(j == 0, k == 0))
    def _():
        stats_ref[...] = jnp.zeros_like(stats_ref)

    stats_ref[...] += jnp.concatenate([s, q], axis=0)


def _ystats(gathered, zc):
    rows, w = gathered.shape
    npts = rows // _K
    jt = npts // _PTILE
    return pl.pallas_call(
        _ystats_body,
        grid=(jt, _K),
        in_specs=[
            pl.BlockSpec((_PTILE, w), lambda j, k: (k * jt + j, 0)),
            pl.BlockSpec((_PTILE, w), lambda j, k: (j, 0)),
        ],
        out_specs=pl.BlockSpec((2, w), lambda j, k: (0, 0)),
        out_shape=jax.ShapeDtypeStruct((2, w), jnp.float32),
    )(gathered, zc)


def _fuse_body(g_ref, zc_ref, stats_ref, gam_ref, bet_ref, wft_ref,
               hmax_ref, hmin_ref, hstats_ref, cnt):
    j = pl.program_id(0)
    k = pl.program_id(1)
    mean = stats_ref[0:1, :] / cnt
    var = stats_ref[1:2, :] / cnt - mean * mean
    scale = gam_ref[...] / jnp.sqrt(var + _EPS)
    shift = bet_ref[...] - mean * scale
    y = g_ref[...] + zc_ref[...]
    a = jnp.maximum(y * scale + shift, 0.0)
    h = lax.dot_general(a, wft_ref[...], (((1,), (0,)), ((), ())),
                        preferred_element_type=jnp.float32)
    hs = jnp.sum(h, axis=0, keepdims=True)
    hq = jnp.sum(h * h, axis=0, keepdims=True)

    @pl.when(jnp.logical_and(j == 0, k == 0))
    def _():
        hstats_ref[...] = jnp.zeros_like(hstats_ref)

    hstats_ref[...] += jnp.concatenate([hs, hq], axis=0)

    @pl.when(k == 0)
    def _():
        hmax_ref[...] = h
        hmin_ref[...] = h

    @pl.when(k > 0)
    def _():
        hmax_ref[...] = jnp.maximum(hmax_ref[...], h)
        hmin_ref[...] = jnp.minimum(hmin_ref[...], h)


def _fuse(gathered, zc, ystats, gamma, beta, wft):
    rows, w = gathered.shape
    npts = rows // _K
    o = wft.shape[1]
    jt = npts // _PTILE
    return pl.pallas_call(
        functools.partial(_fuse_body, cnt=float(rows)),
        grid=(jt, _K),
        in_specs=[
            pl.BlockSpec((_PTILE, w), lambda j, k: (k * jt + j, 0)),
            pl.BlockSpec((_PTILE, w), lambda j, k: (j, 0)),
            pl.BlockSpec((2, w), lambda j, k: (0, 0)),
            pl.BlockSpec((1, w), lambda j, k: (0, 0)),
            pl.BlockSpec((1, w), lambda j, k: (0, 0)),
            pl.BlockSpec((w, o), lambda j, k: (0, 0)),
        ],
        out_specs=[
            pl.BlockSpec((_PTILE, o), lambda j, k: (j, 0)),
            pl.BlockSpec((_PTILE, o), lambda j, k: (j, 0)),
            pl.BlockSpec((2, o), lambda j, k: (0, 0)),
        ],
        out_shape=[
            jax.ShapeDtypeStruct((npts, o), jnp.float32),
            jax.ShapeDtypeStruct((npts, o), jnp.float32),
            jax.ShapeDtypeStruct((2, o), jnp.float32),
        ],
    )(gathered, zc, ystats, gamma, beta, wft)


def _final_body(hmax_ref, hmin_ref, hstats_ref, gam_ref, bet_ref, out_ref, cnt):
    mean = hstats_ref[0:1, :] / cnt
    var = hstats_ref[1:2, :] / cnt - mean * mean
    scale = gam_ref[...] / jnp.sqrt(var + _EPS)
    shift = bet_ref[...] - mean * scale
    pick = jnp.where(scale >= 0.0, hmax_ref[...], hmin_ref[...])
    out_ref[...] = jnp.maximum(pick * scale + shift, 0.0)


def _final(hmax, hmin, hstats, gamma, beta, cnt):
    npts, o = hmax.shape
    tile = 4096
    grid = (npts // tile,)
    return pl.pallas_call(
        functools.partial(_final_body, cnt=cnt),
        grid=grid,
        in_specs=[
            pl.BlockSpec((tile, o), lambda i: (i, 0)),
            pl.BlockSpec((tile, o), lambda i: (i, 0)),
            pl.BlockSpec((2, o), lambda i: (0, 0)),
            pl.BlockSpec((1, o), lambda i: (0, 0)),
            pl.BlockSpec((1, o), lambda i: (0, 0)),
        ],
        out_specs=pl.BlockSpec((tile, o), lambda i: (i, 0)),
        out_shape=jax.ShapeDtypeStruct((npts, o), jnp.float32),
    )(hmax, hmin, hstats, gamma, beta)


def kernel(points, features, W_geom, g_geom, b_geom, W_sem, g_sem, b_sem,
           W_fuse, g_fuse, b_fuse):
    b, n, _ = points.shape
    c = features.shape[-1]
    o = W_geom.shape[0]
    w = 2 * o

    wg2 = W_geom[:, 3:6]
    wgc = W_geom[:, 0:3] - wg2
    ws2 = W_sem[:, c:]
    wsc = W_sem[:, :c] - ws2
    cin = 3 + c
    wt = jnp.zeros((cin, w), jnp.float32)
    wt = wt.at[0:3, 0:o].set(wg2.T).at[3:cin, o:w].set(ws2.T)
    wz = jnp.zeros((cin, w), jnp.float32)
    wz = wz.at[0:3, 0:o].set(wgc.T).at[3:cin, o:w].set(wsc.T)
    wft = W_fuse.T
    gamma = jnp.concatenate([g_geom, g_sem]).reshape(1, w)
    beta = jnp.concatenate([b_geom, b_sem]).reshape(1, w)
    gf = g_fuse.reshape(1, o)
    bf = b_fuse.reshape(1, o)
    points_t = jnp.transpose(points, (0, 2, 1))
    pf = jnp.concatenate([points, features], axis=-1)

    idx = _knn(points, points_t)
    t, zc = _tables(pf, wt, wz)
    idx_km = idx.reshape(b * n, _K).T.reshape(-1)
    gathered = _sc_gather(t, idx_km)
    ystats = _ystats(gathered, zc)
    hmax, hmin, hstats = _fuse(gathered, zc, ystats, gamma, beta, wft)
    out = _final(hmax, hmin, hstats, gf, bf, float(b * n * _K))
    return out.reshape(b, n, o)


# speedup vs baseline: 18.1235x; 1.1606x over previous
import functools

import jax
import jax.numpy as jnp
from jax import lax
from jax.experimental import pallas as pl
from jax.experimental.pallas import tpu as pltpu
from jax.experimental.pallas import tpu_sc as plsc

_EPS = 1e-5
_K = 16
_ROW_TILE = 256
_PTILE = 2048


def _knn_body(prow_ref, pcol_ref, idx_ref):
    b = pl.program_id(0)
    n = pcol_ref.shape[2]
    r = prow_ref.shape[1]
    pr = prow_ref[0]
    pc = pcol_ref[0]
    xr, yr, zr = pr[:, 0:1], pr[:, 1:2], pr[:, 2:3]
    xc, yc, zc = pc[0:1, :], pc[1:2, :], pc[2:3, :]
    sq_r = xr * xr + yr * yr + zr * zr
    sq_c = xc * xc + yc * yc + zc * zc
    rnd = lambda v: v.astype(jnp.bfloat16).astype(jnp.float32)
    cross = (rnd(xr) * rnd(xc) + rnd(yr) * rnd(yc)
             + rnd(zr) * rnd(zc))
    d = (sq_r + sq_c) - 2.0 * cross
    iota = lax.broadcasted_iota(jnp.int32, (r, n), 1).astype(jnp.float32)
    iota16 = lax.broadcasted_iota(jnp.int32, (r, _K), 1).astype(jnp.float32)
    acc = jnp.zeros((r, _K), jnp.float32)
    big = jnp.float32(jnp.inf)
    nf = jnp.float32(n)
    for k in range(_K):
        m = jnp.min(d, axis=1, keepdims=True)
        am = jnp.min(jnp.where(d <= m, iota, nf), axis=1, keepdims=True)
        acc = jnp.where(iota16 == k, am, acc)
        d = jnp.where(iota == am, big, d)
    idx_ref[0] = acc.astype(jnp.int32) + b * n


def _knn(points, points_t):
    b, n, _ = points.shape
    grid = (b, n // _ROW_TILE)
    return pl.pallas_call(
        _knn_body,
        grid=grid,
        in_specs=[
            pl.BlockSpec((1, _ROW_TILE, 3), lambda bi, i: (bi, i, 0)),
            pl.BlockSpec((1, 3, n), lambda bi, i: (bi, 0, 0)),
        ],
        out_specs=pl.BlockSpec((1, _ROW_TILE, _K), lambda bi, i: (bi, i, 0)),
        out_shape=jax.ShapeDtypeStruct((b, n, _K), jnp.int32),
    )(points, points_t)


def _tables_body(pf_ref, wt_ref, wz_ref, t_ref, zc_ref):
    pf = pf_ref[0]
    dn = (((1,), (0,)), ((), ()))
    t_ref[0] = lax.dot_general(pf, wt_ref[...], dn,
                               preferred_element_type=jnp.float32)
    zc_ref[0] = lax.dot_general(pf, wz_ref[...], dn,
                                preferred_element_type=jnp.float32)


def _tables(pf, wt, wz):
    b, n, cin = pf.shape
    w = wt.shape[1]
    t, zc = pl.pallas_call(
        _tables_body,
        grid=(b,),
        in_specs=[
            pl.BlockSpec((1, n, cin), lambda bi: (bi, 0, 0)),
            pl.BlockSpec((cin, w), lambda bi: (0, 0)),
            pl.BlockSpec((cin, w), lambda bi: (0, 0)),
        ],
        out_specs=[
            pl.BlockSpec((1, n, w), lambda bi: (bi, 0, 0)),
            pl.BlockSpec((1, n, w), lambda bi: (bi, 0, 0)),
        ],
        out_shape=[
            jax.ShapeDtypeStruct((b, n, w), jnp.float32),
            jax.ShapeDtypeStruct((b, n, w), jnp.float32),
        ],
    )(pf, wt, wz)
    return t.reshape(b * n, w), zc.reshape(b * n, w)


_GCH = 128


def _sc_gather(table, idx_flat):
    rows, width = idx_flat.shape[0], table.shape[1]
    info = plsc.get_sparse_core_info()
    nw = info.num_cores * info.num_subcores
    per_w = rows // nw
    n_ch = per_w // _GCH
    idx2d = idx_flat.reshape(nw * n_ch, _GCH)
    mesh = plsc.VectorSubcoreMesh(core_axis_name="c", subcore_axis_name="s")

    @functools.partial(
        pl.kernel,
        out_type=jax.ShapeDtypeStruct((rows, width), jnp.float32),
        mesh=mesh,
        scratch_types=[
            pltpu.VMEM((n_ch, _GCH), jnp.int32),
            pltpu.VMEM((_GCH, width), jnp.float32),
            pltpu.VMEM((_GCH, width), jnp.float32),
            pltpu.SemaphoreType.DMA,
            pltpu.SemaphoreType.DMA,
        ],
    )
    def gather_kernel(tbl_hbm, idx_hbm, out_hbm, idx_v, buf0, buf1, sem0, sem1):
        wid = lax.axis_index("s") * info.num_cores + lax.axis_index("c")
        pltpu.sync_copy(idx_hbm.at[pl.ds(wid * n_ch, n_ch)], idx_v)
        base = wid * per_w
        pltpu.async_copy(tbl_hbm.at[idx_v.at[0]], buf0, sem0)

        def body(j2, _):
            j = 2 * j2
            pltpu.async_copy(tbl_hbm.at[idx_v.at[j + 1]], buf1, sem1)
            pltpu.make_async_copy(tbl_hbm.at[idx_v.at[j]], buf0, sem0).wait()
            pltpu.sync_copy(buf0, out_hbm.at[pl.ds(base + j * _GCH, _GCH)])

            @pl.when(j + 2 < n_ch)
            def _():
                pltpu.async_copy(tbl_hbm.at[idx_v.at[j + 2]], buf0, sem0)

            pltpu.make_async_copy(tbl_hbm.at[idx_v.at[j + 1]], buf1,
                                  sem1).wait()
            pltpu.sync_copy(buf1,
                            out_hbm.at[pl.ds(base + (j + 1) * _GCH, _GCH)])
            return 0

        lax.fori_loop(0, n_ch // 2, body, 0)

    return gather_kernel(table, idx2d)


def _ystats_body(g_ref, zc_ref, stats_ref):
    j = pl.program_id(0)
    k = pl.program_id(1)
    y = g_ref[...] + zc_ref[...]
    s = jnp.sum(y, axis=0, keepdims=True)
    q = jnp.sum(y * y, axis=0, keepdims=True)

    @pl.when(jnp.logical_and(j == 0, k == 0))
    def _():
        stats_ref[...] = jnp.zeros_like(stats_ref)

    stats_ref[...] += jnp.concatenate([s, q], axis=0)


def _ystats(gathered, zc):
    rows, w = gathered.shape
    npts = rows // _K
    jt = npts // _PTILE
    return pl.pallas_call(
        _ystats_body,
        grid=(jt, _K),
        in_specs=[
            pl.BlockSpec((_PTILE, w), lambda j, k: (k * jt + j, 0)),
            pl.BlockSpec((_PTILE, w), lambda j, k: (j, 0)),
        ],
        out_specs=pl.BlockSpec((2, w), lambda j, k: (0, 0)),
        out_shape=jax.ShapeDtypeStruct((2, w), jnp.float32),
    )(gathered, zc)


def _fuse_body(g_ref, zc_ref, stats_ref, gam_ref, bet_ref, wft_ref,
               hmax_ref, hmin_ref, hstats_ref, cnt):
    j = pl.program_id(0)
    k = pl.program_id(1)
    mean = stats_ref[0:1, :] / cnt
    var = stats_ref[1:2, :] / cnt - mean * mean
    scale = gam_ref[...] / jnp.sqrt(var + _EPS)
    shift = bet_ref[...] - mean * scale
    y = g_ref[...] + zc_ref[...]
    a = jnp.maximum(y * scale + shift, 0.0)
    h = lax.dot_general(a, wft_ref[...], (((1,), (0,)), ((), ())),
                        preferred_element_type=jnp.float32)
    hs = jnp.sum(h, axis=0, keepdims=True)
    hq = jnp.sum(h * h, axis=0, keepdims=True)

    @pl.when(jnp.logical_and(j == 0, k == 0))
    def _():
        hstats_ref[...] = jnp.zeros_like(hstats_ref)

    hstats_ref[...] += jnp.concatenate([hs, hq], axis=0)

    @pl.when(k == 0)
    def _():
        hmax_ref[...] = h
        hmin_ref[...] = h

    @pl.when(k > 0)
    def _():
        hmax_ref[...] = jnp.maximum(hmax_ref[...], h)
        hmin_ref[...] = jnp.minimum(hmin_ref[...], h)


def _fuse(gathered, zc, ystats, gamma, beta, wft):
    rows, w = gathered.shape
    npts = rows // _K
    o = wft.shape[1]
    jt = npts // _PTILE
    return pl.pallas_call(
        functools.partial(_fuse_body, cnt=float(rows)),
        grid=(jt, _K),
        in_specs=[
            pl.BlockSpec((_PTILE, w), lambda j, k: (k * jt + j, 0)),
            pl.BlockSpec((_PTILE, w), lambda j, k: (j, 0)),
            pl.BlockSpec((2, w), lambda j, k: (0, 0)),
            pl.BlockSpec((1, w), lambda j, k: (0, 0)),
            pl.BlockSpec((1, w), lambda j, k: (0, 0)),
            pl.BlockSpec((w, o), lambda j, k: (0, 0)),
        ],
        out_specs=[
            pl.BlockSpec((_PTILE, o), lambda j, k: (j, 0)),
            pl.BlockSpec((_PTILE, o), lambda j, k: (j, 0)),
            pl.BlockSpec((2, o), lambda j, k: (0, 0)),
        ],
        out_shape=[
            jax.ShapeDtypeStruct((npts, o), jnp.float32),
            jax.ShapeDtypeStruct((npts, o), jnp.float32),
            jax.ShapeDtypeStruct((2, o), jnp.float32),
        ],
    )(gathered, zc, ystats, gamma, beta, wft)


def _final_body(hmax_ref, hmin_ref, hstats_ref, gam_ref, bet_ref, out_ref, cnt):
    mean = hstats_ref[0:1, :] / cnt
    var = hstats_ref[1:2, :] / cnt - mean * mean
    scale = gam_ref[...] / jnp.sqrt(var + _EPS)
    shift = bet_ref[...] - mean * scale
    pick = jnp.where(scale >= 0.0, hmax_ref[...], hmin_ref[...])
    out_ref[...] = jnp.maximum(pick * scale + shift, 0.0)


def _final(hmax, hmin, hstats, gamma, beta, cnt):
    npts, o = hmax.shape
    tile = 4096
    grid = (npts // tile,)
    return pl.pallas_call(
        functools.partial(_final_body, cnt=cnt),
        grid=grid,
        in_specs=[
            pl.BlockSpec((tile, o), lambda i: (i, 0)),
            pl.BlockSpec((tile, o), lambda i: (i, 0)),
            pl.BlockSpec((2, o), lambda i: (0, 0)),
            pl.BlockSpec((1, o), lambda i: (0, 0)),
            pl.BlockSpec((1, o), lambda i: (0, 0)),
        ],
        out_specs=pl.BlockSpec((tile, o), lambda i: (i, 0)),
        out_shape=jax.ShapeDtypeStruct((npts, o), jnp.float32),
    )(hmax, hmin, hstats, gamma, beta)


def kernel(points, features, W_geom, g_geom, b_geom, W_sem, g_sem, b_sem,
           W_fuse, g_fuse, b_fuse):
    b, n, _ = points.shape
    c = features.shape[-1]
    o = W_geom.shape[0]
    w = 2 * o

    wg2 = W_geom[:, 3:6]
    wgc = W_geom[:, 0:3] - wg2
    ws2 = W_sem[:, c:]
    wsc = W_sem[:, :c] - ws2
    cin = 3 + c
    wt = jnp.zeros((cin, w), jnp.float32)
    wt = wt.at[0:3, 0:o].set(wg2.T).at[3:cin, o:w].set(ws2.T)
    wz = jnp.zeros((cin, w), jnp.float32)
    wz = wz.at[0:3, 0:o].set(wgc.T).at[3:cin, o:w].set(wsc.T)
    wft = W_fuse.T
    gamma = jnp.concatenate([g_geom, g_sem]).reshape(1, w)
    beta = jnp.concatenate([b_geom, b_sem]).reshape(1, w)
    gf = g_fuse.reshape(1, o)
    bf = b_fuse.reshape(1, o)
    points_t = jnp.transpose(points, (0, 2, 1))
    pf = jnp.concatenate([points, features], axis=-1)

    idx = _knn(points, points_t)
    t, zc = _tables(pf, wt, wz)
    idx_km = idx.reshape(b * n, _K).T.reshape(-1)
    gathered = _sc_gather(t, idx_km)
    ystats = _ystats(gathered, zc)
    hmax, hmin, hstats = _fuse(gathered, zc, ystats, gamma, beta, wft)
    out = _final(hmax, hmin, hstats, gf, bf, float(b * n * _K))
    return out.reshape(b, n, o)


# speedup vs baseline: 18.2280x; 1.0058x over previous
import functools

import jax
import jax.numpy as jnp
from jax import lax
from jax.experimental import pallas as pl
from jax.experimental.pallas import tpu as pltpu
from jax.experimental.pallas import tpu_sc as plsc

_EPS = 1e-5
_K = 16
_ROW_TILE = 256
_PTILE = 2048


def _knn_body(prow_ref, pcol_ref, idx_ref):
    b = pl.program_id(0)
    n = pcol_ref.shape[2]
    r = prow_ref.shape[1]
    pr = prow_ref[0]
    pc = pcol_ref[0]
    xr, yr, zr = pr[:, 0:1], pr[:, 1:2], pr[:, 2:3]
    xc, yc, zc = pc[0:1, :], pc[1:2, :], pc[2:3, :]
    sq_r = xr * xr + yr * yr + zr * zr
    sq_c = xc * xc + yc * yc + zc * zc
    rnd = lambda v: v.astype(jnp.bfloat16).astype(jnp.float32)
    cross = (rnd(xr) * rnd(xc) + rnd(yr) * rnd(yc)
             + rnd(zr) * rnd(zc))
    d = (sq_r + sq_c) - 2.0 * cross
    h = n // 2
    iota = lax.broadcasted_iota(jnp.int32, (r, h), 1).astype(jnp.float32)
    iota16 = lax.broadcasted_iota(jnp.int32, (r, _K), 1).astype(jnp.float32)
    acc = jnp.zeros((r, _K), jnp.float32)
    big = jnp.float32(jnp.inf)
    dl = d[:, 0:h]
    dr = d[:, h:n]
    c = dr < dl
    v1 = jnp.minimum(dl, dr)
    v2 = jnp.maximum(dl, dr)
    i1 = jnp.where(c, iota + h, iota)
    i2 = jnp.where(c, iota, iota + h)
    for k in range(_K):
        m = jnp.min(v1, axis=1, keepdims=True)
        am = jnp.min(jnp.where(v1 <= m, i1, big), axis=1, keepdims=True)
        acc = jnp.where(iota16 == k, am, acc)
        msk = i1 == am
        v1 = jnp.where(msk, v2, v1)
        i1 = jnp.where(msk, i2, i1)
        v2 = jnp.where(msk, big, v2)
    idx_ref[0] = acc.astype(jnp.int32) + b * n


def _knn(points, points_t):
    b, n, _ = points.shape
    grid = (b, n // _ROW_TILE)
    return pl.pallas_call(
        _knn_body,
        grid=grid,
        in_specs=[
            pl.BlockSpec((1, _ROW_TILE, 3), lambda bi, i: (bi, i, 0)),
            pl.BlockSpec((1, 3, n), lambda bi, i: (bi, 0, 0)),
        ],
        out_specs=pl.BlockSpec((1, _ROW_TILE, _K), lambda bi, i: (bi, i, 0)),
        out_shape=jax.ShapeDtypeStruct((b, n, _K), jnp.int32),
    )(points, points_t)


def _tables_body(pf_ref, wt_ref, wz_ref, t_ref, zc_ref):
    pf = pf_ref[0]
    dn = (((1,), (0,)), ((), ()))
    t_ref[0] = lax.dot_general(pf, wt_ref[...], dn,
                               preferred_element_type=jnp.float32)
    zc_ref[0] = lax.dot_general(pf, wz_ref[...], dn,
                                preferred_element_type=jnp.float32)


def _tables(pf, wt, wz):
    b, n, cin = pf.shape
    w = wt.shape[1]
    t, zc = pl.pallas_call(
        _tables_body,
        grid=(b,),
        in_specs=[
            pl.BlockSpec((1, n, cin), lambda bi: (bi, 0, 0)),
            pl.BlockSpec((cin, w), lambda bi: (0, 0)),
            pl.BlockSpec((cin, w), lambda bi: (0, 0)),
        ],
        out_specs=[
            pl.BlockSpec((1, n, w), lambda bi: (bi, 0, 0)),
            pl.BlockSpec((1, n, w), lambda bi: (bi, 0, 0)),
        ],
        out_shape=[
            jax.ShapeDtypeStruct((b, n, w), jnp.float32),
            jax.ShapeDtypeStruct((b, n, w), jnp.float32),
        ],
    )(pf, wt, wz)
    return t.reshape(b * n, w), zc.reshape(b * n, w)


_GCH = 128


def _sc_gather(table, idx_flat):
    rows, width = idx_flat.shape[0], table.shape[1]
    dt = table.dtype
    info = plsc.get_sparse_core_info()
    nw = info.num_cores * info.num_subcores
    per_w = rows // nw
    n_ch = per_w // _GCH
    idx2d = idx_flat.reshape(nw * n_ch, _GCH)
    mesh = plsc.VectorSubcoreMesh(core_axis_name="c", subcore_axis_name="s")

    @functools.partial(
        pl.kernel,
        out_type=jax.ShapeDtypeStruct((rows, width), dt),
        mesh=mesh,
        scratch_types=[
            pltpu.VMEM((n_ch, _GCH), jnp.int32),
            pltpu.VMEM((_GCH, width), dt),
            pltpu.VMEM((_GCH, width), dt),
            pltpu.SemaphoreType.DMA,
            pltpu.SemaphoreType.DMA,
        ],
    )
    def gather_kernel(tbl_hbm, idx_hbm, out_hbm, idx_v, buf0, buf1, sem0, sem1):
        wid = lax.axis_index("s") * info.num_cores + lax.axis_index("c")
        pltpu.sync_copy(idx_hbm.at[pl.ds(wid * n_ch, n_ch)], idx_v)
        base = wid * per_w
        pltpu.async_copy(tbl_hbm.at[idx_v.at[0]], buf0, sem0)

        def body(j2, _):
            j = 2 * j2
            pltpu.async_copy(tbl_hbm.at[idx_v.at[j + 1]], buf1, sem1)
            pltpu.make_async_copy(tbl_hbm.at[idx_v.at[j]], buf0, sem0).wait()
            pltpu.sync_copy(buf0, out_hbm.at[pl.ds(base + j * _GCH, _GCH)])

            @pl.when(j + 2 < n_ch)
            def _():
                pltpu.async_copy(tbl_hbm.at[idx_v.at[j + 2]], buf0, sem0)

            pltpu.make_async_copy(tbl_hbm.at[idx_v.at[j + 1]], buf1,
                                  sem1).wait()
            pltpu.sync_copy(buf1,
                            out_hbm.at[pl.ds(base + (j + 1) * _GCH, _GCH)])
            return 0

        lax.fori_loop(0, n_ch // 2, body, 0)

    return gather_kernel(table, idx2d)


def _stats_fuse_body(g_ref, zc_ref, gam_ref, bet_ref, wft_ref,
                     hmax_ref, hmin_ref, hstats_ref, ystats_ref, cnt):
    p = pl.program_id(0)
    j = pl.program_id(1)
    k = pl.program_id(2)
    first = jnp.logical_and(j == 0, k == 0)
    y = g_ref[...].astype(jnp.float32) + zc_ref[...]

    @pl.when(p == 0)
    def _():
        @pl.when(first)
        def _():
            ystats_ref[...] = jnp.zeros_like(ystats_ref)

        s = jnp.sum(y, axis=0, keepdims=True)
        q = jnp.sum(y * y, axis=0, keepdims=True)
        ystats_ref[...] += jnp.concatenate([s, q], axis=0)

    @pl.when(p == 1)
    def _():
        mean = ystats_ref[0:1, :] / cnt
        var = ystats_ref[1:2, :] / cnt - mean * mean
        scale = gam_ref[...] / jnp.sqrt(var + _EPS)
        shift = bet_ref[...] - mean * scale
        a = jnp.maximum(y * scale + shift, 0.0).astype(jnp.bfloat16)
        h = lax.dot_general(a, wft_ref[...].astype(jnp.bfloat16),
                            (((1,), (0,)), ((), ())),
                            preferred_element_type=jnp.float32)
        hs = jnp.sum(h, axis=0, keepdims=True)
        hq = jnp.sum(h * h, axis=0, keepdims=True)

        @pl.when(first)
        def _():
            hstats_ref[...] = jnp.zeros_like(hstats_ref)

        hstats_ref[...] += jnp.concatenate([hs, hq], axis=0)

        @pl.when(k == 0)
        def _():
            hmax_ref[...] = h
            hmin_ref[...] = h

        @pl.when(k > 0)
        def _():
            hmax_ref[...] = jnp.maximum(hmax_ref[...], h)
            hmin_ref[...] = jnp.minimum(hmin_ref[...], h)


def _stats_fuse(gathered, zc, gamma, beta, wft):
    rows, w = gathered.shape
    npts = rows // _K
    o = wft.shape[1]
    jt = npts // _PTILE
    return pl.pallas_call(
        functools.partial(_stats_fuse_body, cnt=float(rows)),
        grid=(2, jt, _K),
        in_specs=[
            pl.BlockSpec((_PTILE, w), lambda p, j, k: (k * jt + j, 0)),
            pl.BlockSpec((_PTILE, w), lambda p, j, k: (j, 0)),
            pl.BlockSpec((1, w), lambda p, j, k: (0, 0)),
            pl.BlockSpec((1, w), lambda p, j, k: (0, 0)),
            pl.BlockSpec((w, o), lambda p, j, k: (0, 0)),
        ],
        out_specs=[
            pl.BlockSpec((_PTILE, o), lambda p, j, k: (j, 0)),
            pl.BlockSpec((_PTILE, o), lambda p, j, k: (j, 0)),
            pl.BlockSpec((2, o), lambda p, j, k: (0, 0)),
        ],
        out_shape=[
            jax.ShapeDtypeStruct((npts, o), jnp.float32),
            jax.ShapeDtypeStruct((npts, o), jnp.float32),
            jax.ShapeDtypeStruct((2, o), jnp.float32),
        ],
        scratch_shapes=[pltpu.VMEM((2, w), jnp.float32)],
    )(gathered, zc, gamma, beta, wft)


def _final_body(hmax_ref, hmin_ref, hstats_ref, gam_ref, bet_ref, out_ref, cnt):
    mean = hstats_ref[0:1, :] / cnt
    var = hstats_ref[1:2, :] / cnt - mean * mean
    scale = gam_ref[...] / jnp.sqrt(var + _EPS)
    shift = bet_ref[...] - mean * scale
    pick = jnp.where(scale >= 0.0, hmax_ref[...], hmin_ref[...])
    out_ref[...] = jnp.maximum(pick * scale + shift, 0.0)


def _final(hmax, hmin, hstats, gamma, beta, cnt):
    npts, o = hmax.shape
    tile = 4096
    grid = (npts // tile,)
    return pl.pallas_call(
        functools.partial(_final_body, cnt=cnt),
        grid=grid,
        in_specs=[
            pl.BlockSpec((tile, o), lambda i: (i, 0)),
            pl.BlockSpec((tile, o), lambda i: (i, 0)),
            pl.BlockSpec((2, o), lambda i: (0, 0)),
            pl.BlockSpec((1, o), lambda i: (0, 0)),
            pl.BlockSpec((1, o), lambda i: (0, 0)),
        ],
        out_specs=pl.BlockSpec((tile, o), lambda i: (i, 0)),
        out_shape=jax.ShapeDtypeStruct((npts, o), jnp.float32),
    )(hmax, hmin, hstats, gamma, beta)


def kernel(points, features, W_geom, g_geom, b_geom, W_sem, g_sem, b_sem,
           W_fuse, g_fuse, b_fuse):
    b, n, _ = points.shape
    c = features.shape[-1]
    o = W_geom.shape[0]
    w = 2 * o

    wg2 = W_geom[:, 3:6]
    wgc = W_geom[:, 0:3] - wg2
    ws2 = W_sem[:, c:]
    wsc = W_sem[:, :c] - ws2
    cin = 3 + c
    wt = jnp.zeros((cin, w), jnp.float32)
    wt = wt.at[0:3, 0:o].set(wg2.T).at[3:cin, o:w].set(ws2.T)
    wz = jnp.zeros((cin, w), jnp.float32)
    wz = wz.at[0:3, 0:o].set(wgc.T).at[3:cin, o:w].set(wsc.T)
    wft = W_fuse.T
    gamma = jnp.concatenate([g_geom, g_sem]).reshape(1, w)
    beta = jnp.concatenate([b_geom, b_sem]).reshape(1, w)
    gf = g_fuse.reshape(1, o)
    bf = b_fuse.reshape(1, o)
    points_t = jnp.transpose(points, (0, 2, 1))
    pf = jnp.concatenate([points, features], axis=-1)

    idx = _knn(points, points_t)
    t, zc = _tables(pf, wt, wz)
    idx_km = idx.reshape(b * n, _K).T.reshape(-1)
    gathered = _sc_gather(t, idx_km)
    hmax, hmin, hstats = _stats_fuse(gathered, zc, gamma, beta, wft)
    out = _final(hmax, hmin, hstats, gf, bf, float(b * n * _K))
    return out.reshape(b, n, o)


# speedup vs baseline: 22.9044x; 1.2565x over previous
import functools

import jax
import jax.numpy as jnp
from jax import lax
from jax.experimental import pallas as pl
from jax.experimental.pallas import tpu as pltpu
from jax.experimental.pallas import tpu_sc as plsc

_EPS = 1e-5
_K = 16
_ROW_TILE = 512
_PTILE = 2048


_DEPTH = 5
_LW = 128


def _knn_body(prow_ref, pcol_ref, idx_ref):
    b = pl.program_id(0)
    n = pcol_ref.shape[2]
    r = prow_ref.shape[1]
    nch = n // _LW
    pr = prow_ref[0]
    pc = pcol_ref[0]
    xr, yr, zr = pr[:, 0:1], pr[:, 1:2], pr[:, 2:3]
    xc, yc, zc = pc[0:1, :], pc[1:2, :], pc[2:3, :]
    sq_r = xr * xr + yr * yr + zr * zr
    sq_c = xc * xc + yc * yc + zc * zc
    rnd = lambda v: v.astype(jnp.bfloat16).astype(jnp.float32)
    xrr, yrr, zrr = rnd(xr), rnd(yr), rnd(zr)
    xcr, ycr, zcr = rnd(xc), rnd(yc), rnd(zc)
    big = jnp.float32(jnp.inf)
    iota = lax.broadcasted_iota(jnp.int32, (r, _LW), 1).astype(jnp.float32)
    iota16 = lax.broadcasted_iota(jnp.int32, (r, _K), 1).astype(jnp.float32)
    vs = [jnp.full((r, _LW), big) for _ in range(_DEPTH)]
    ds = [jnp.zeros((r, _LW), jnp.float32) for _ in range(_DEPTH)]
    for c in range(nch):
        sl = slice(c * _LW, (c + 1) * _LW)
        cross = (xrr * xcr[:, sl] + yrr * ycr[:, sl] + zrr * zcr[:, sl])
        tv = (sq_r + sq_c[:, sl]) - 2.0 * cross
        ti = iota + jnp.float32(c * _LW)
        for j in range(_DEPTH):
            cj = tv < vs[j]
            nv = jnp.where(cj, tv, vs[j])
            tv = jnp.where(cj, vs[j], tv)
            ni = jnp.where(cj, ti, ds[j])
            ti = jnp.where(cj, ds[j], ti)
            vs[j] = nv
            ds[j] = ni
    acc = jnp.zeros((r, _K), jnp.float32)
    for k in range(_K):
        m = jnp.min(vs[0], axis=1, keepdims=True)
        am = jnp.min(jnp.where(vs[0] <= m, ds[0], big), axis=1, keepdims=True)
        acc = jnp.where(iota16 == k, am, acc)
        msk = ds[0] == am
        for j in range(_DEPTH - 1):
            vs[j] = jnp.where(msk, vs[j + 1], vs[j])
            ds[j] = jnp.where(msk, ds[j + 1], ds[j])
        vs[_DEPTH - 1] = jnp.where(msk, big, vs[_DEPTH - 1])
    idx_ref[0] = acc.astype(jnp.int32) + b * n


def _knn(points, points_t):
    b, n, _ = points.shape
    grid = (b, n // _ROW_TILE)
    return pl.pallas_call(
        _knn_body,
        grid=grid,
        in_specs=[
            pl.BlockSpec((1, _ROW_TILE, 3), lambda bi, i: (bi, i, 0)),
            pl.BlockSpec((1, 3, n), lambda bi, i: (bi, 0, 0)),
        ],
        out_specs=pl.BlockSpec((1, _ROW_TILE, _K), lambda bi, i: (bi, i, 0)),
        out_shape=jax.ShapeDtypeStruct((b, n, _K), jnp.int32),
    )(points, points_t)


def _tables_body(pf_ref, wt_ref, wz_ref, t_ref, zc_ref):
    pf = pf_ref[0]
    dn = (((1,), (0,)), ((), ()))
    t_ref[0] = lax.dot_general(pf, wt_ref[...], dn,
                               preferred_element_type=jnp.float32)
    zc_ref[0] = lax.dot_general(pf, wz_ref[...], dn,
                                preferred_element_type=jnp.float32)


def _tables(pf, wt, wz):
    b, n, cin = pf.shape
    w = wt.shape[1]
    t, zc = pl.pallas_call(
        _tables_body,
        grid=(b,),
        in_specs=[
            pl.BlockSpec((1, n, cin), lambda bi: (bi, 0, 0)),
            pl.BlockSpec((cin, w), lambda bi: (0, 0)),
            pl.BlockSpec((cin, w), lambda bi: (0, 0)),
        ],
        out_specs=[
            pl.BlockSpec((1, n, w), lambda bi: (bi, 0, 0)),
            pl.BlockSpec((1, n, w), lambda bi: (bi, 0, 0)),
        ],
        out_shape=[
            jax.ShapeDtypeStruct((b, n, w), jnp.float32),
            jax.ShapeDtypeStruct((b, n, w), jnp.float32),
        ],
    )(pf, wt, wz)
    return t.reshape(b * n, w), zc.reshape(b * n, w)


_GCH = 128


def _sc_gather(table, idx_flat):
    rows, width = idx_flat.shape[0], table.shape[1]
    dt = table.dtype
    info = plsc.get_sparse_core_info()
    nw = info.num_cores * info.num_subcores
    per_w = rows // nw
    n_ch = per_w // _GCH
    idx2d = idx_flat.reshape(nw * n_ch, _GCH)
    mesh = plsc.VectorSubcoreMesh(core_axis_name="c", subcore_axis_name="s")

    @functools.partial(
        pl.kernel,
        out_type=jax.ShapeDtypeStruct((rows, width), dt),
        mesh=mesh,
        scratch_types=[
            pltpu.VMEM((n_ch, _GCH), jnp.int32),
            pltpu.VMEM((_GCH, width), dt),
            pltpu.VMEM((_GCH, width), dt),
            pltpu.SemaphoreType.DMA,
            pltpu.SemaphoreType.DMA,
        ],
    )
    def gather_kernel(tbl_hbm, idx_hbm, out_hbm, idx_v, buf0, buf1, sem0, sem1):
        wid = lax.axis_index("s") * info.num_cores + lax.axis_index("c")
        pltpu.sync_copy(idx_hbm.at[pl.ds(wid * n_ch, n_ch)], idx_v)
        base = wid * per_w
        pltpu.async_copy(tbl_hbm.at[idx_v.at[0]], buf0, sem0)

        def body(j2, _):
            j = 2 * j2
            pltpu.async_copy(tbl_hbm.at[idx_v.at[j + 1]], buf1, sem1)
            pltpu.make_async_copy(tbl_hbm.at[idx_v.at[j]], buf0, sem0).wait()
            pltpu.sync_copy(buf0, out_hbm.at[pl.ds(base + j * _GCH, _GCH)])

            @pl.when(j + 2 < n_ch)
            def _():
                pltpu.async_copy(tbl_hbm.at[idx_v.at[j + 2]], buf0, sem0)

            pltpu.make_async_copy(tbl_hbm.at[idx_v.at[j + 1]], buf1,
                                  sem1).wait()
            pltpu.sync_copy(buf1,
                            out_hbm.at[pl.ds(base + (j + 1) * _GCH, _GCH)])
            return 0

        lax.fori_loop(0, n_ch // 2, body, 0)

    return gather_kernel(table, idx2d)


def _stats_fuse_body(g_ref, zc_ref, gam_ref, bet_ref, wft_ref,
                     hmax_ref, hmin_ref, hstats_ref, ystats_ref, cnt):
    p = pl.program_id(0)
    j = pl.program_id(1)
    k = pl.program_id(2)
    first = jnp.logical_and(j == 0, k == 0)
    y = g_ref[...].astype(jnp.float32) + zc_ref[...]

    @pl.when(p == 0)
    def _():
        @pl.when(first)
        def _():
            ystats_ref[...] = jnp.zeros_like(ystats_ref)

        s = jnp.sum(y, axis=0, keepdims=True)
        q = jnp.sum(y * y, axis=0, keepdims=True)
        ystats_ref[...] += jnp.concatenate([s, q], axis=0)

    @pl.when(p == 1)
    def _():
        mean = ystats_ref[0:1, :] / cnt
        var = ystats_ref[1:2, :] / cnt - mean * mean
        scale = gam_ref[...] / jnp.sqrt(var + _EPS)
        shift = bet_ref[...] - mean * scale
        a = jnp.maximum(y * scale + shift, 0.0).astype(jnp.bfloat16)
        h = lax.dot_general(a, wft_ref[...].astype(jnp.bfloat16),
                            (((1,), (0,)), ((), ())),
                            preferred_element_type=jnp.float32)
        hs = jnp.sum(h, axis=0, keepdims=True)
        hq = jnp.sum(h * h, axis=0, keepdims=True)

        @pl.when(first)
        def _():
            hstats_ref[...] = jnp.zeros_like(hstats_ref)

        hstats_ref[...] += jnp.concatenate([hs, hq], axis=0)

        @pl.when(k == 0)
        def _():
            hmax_ref[...] = h
            hmin_ref[...] = h

        @pl.when(k > 0)
        def _():
            hmax_ref[...] = jnp.maximum(hmax_ref[...], h)
            hmin_ref[...] = jnp.minimum(hmin_ref[...], h)


def _stats_fuse(gathered, zc, gamma, beta, wft):
    rows, w = gathered.shape
    npts = rows // _K
    o = wft.shape[1]
    jt = npts // _PTILE
    return pl.pallas_call(
        functools.partial(_stats_fuse_body, cnt=float(rows)),
        grid=(2, jt, _K),
        in_specs=[
            pl.BlockSpec((_PTILE, w), lambda p, j, k: (k * jt + j, 0)),
            pl.BlockSpec((_PTILE, w), lambda p, j, k: (j, 0)),
            pl.BlockSpec((1, w), lambda p, j, k: (0, 0)),
            pl.BlockSpec((1, w), lambda p, j, k: (0, 0)),
            pl.BlockSpec((w, o), lambda p, j, k: (0, 0)),
        ],
        out_specs=[
            pl.BlockSpec((_PTILE, o), lambda p, j, k: (j, 0)),
            pl.BlockSpec((_PTILE, o), lambda p, j, k: (j, 0)),
            pl.BlockSpec((2, o), lambda p, j, k: (0, 0)),
        ],
        out_shape=[
            jax.ShapeDtypeStruct((npts, o), jnp.float32),
            jax.ShapeDtypeStruct((npts, o), jnp.float32),
            jax.ShapeDtypeStruct((2, o), jnp.float32),
        ],
        scratch_shapes=[pltpu.VMEM((2, w), jnp.float32)],
    )(gathered, zc, gamma, beta, wft)


def _final_body(hmax_ref, hmin_ref, hstats_ref, gam_ref, bet_ref, out_ref, cnt):
    mean = hstats_ref[0:1, :] / cnt
    var = hstats_ref[1:2, :] / cnt - mean * mean
    scale = gam_ref[...] / jnp.sqrt(var + _EPS)
    shift = bet_ref[...] - mean * scale
    pick = jnp.where(scale >= 0.0, hmax_ref[...], hmin_ref[...])
    out_ref[...] = jnp.maximum(pick * scale + shift, 0.0)


def _final(hmax, hmin, hstats, gamma, beta, cnt):
    npts, o = hmax.shape
    tile = 4096
    grid = (npts // tile,)
    return pl.pallas_call(
        functools.partial(_final_body, cnt=cnt),
        grid=grid,
        in_specs=[
            pl.BlockSpec((tile, o), lambda i: (i, 0)),
            pl.BlockSpec((tile, o), lambda i: (i, 0)),
            pl.BlockSpec((2, o), lambda i: (0, 0)),
            pl.BlockSpec((1, o), lambda i: (0, 0)),
            pl.BlockSpec((1, o), lambda i: (0, 0)),
        ],
        out_specs=pl.BlockSpec((tile, o), lambda i: (i, 0)),
        out_shape=jax.ShapeDtypeStruct((npts, o), jnp.float32),
    )(hmax, hmin, hstats, gamma, beta)


def kernel(points, features, W_geom, g_geom, b_geom, W_sem, g_sem, b_sem,
           W_fuse, g_fuse, b_fuse):
    b, n, _ = points.shape
    c = features.shape[-1]
    o = W_geom.shape[0]
    w = 2 * o

    wg2 = W_geom[:, 3:6]
    wgc = W_geom[:, 0:3] - wg2
    ws2 = W_sem[:, c:]
    wsc = W_sem[:, :c] - ws2
    cin = 3 + c
    wt = jnp.zeros((cin, w), jnp.float32)
    wt = wt.at[0:3, 0:o].set(wg2.T).at[3:cin, o:w].set(ws2.T)
    wz = jnp.zeros((cin, w), jnp.float32)
    wz = wz.at[0:3, 0:o].set(wgc.T).at[3:cin, o:w].set(wsc.T)
    wft = W_fuse.T
    gamma = jnp.concatenate([g_geom, g_sem]).reshape(1, w)
    beta = jnp.concatenate([b_geom, b_sem]).reshape(1, w)
    gf = g_fuse.reshape(1, o)
    bf = b_fuse.reshape(1, o)
    points_t = jnp.transpose(points, (0, 2, 1))
    pf = jnp.concatenate([points, features], axis=-1)

    idx = _knn(points, points_t)
    t, zc = _tables(pf, wt, wz)
    idx_km = idx.reshape(b * n, _K).T.reshape(-1)
    gathered = _sc_gather(t, idx_km)
    hmax, hmin, hstats = _stats_fuse(gathered, zc, gamma, beta, wft)
    out = _final(hmax, hmin, hstats, gf, bf, float(b * n * _K))
    return out.reshape(b, n, o)
